# baseline (speedup 1.0000x reference)
import functools

import jax
import jax.numpy as jnp
from jax import lax
from jax.experimental import pallas as pl
from jax.experimental.pallas import tpu as pltpu
from jax.experimental.pallas import tpu_sc as plsc

N, D = 50000, 256
B = 128
NSC = 20000
CHUNK = 80
NCHUNKS = NSC // CHUNK
NC, NS = 2, 16
NW = NC * NS
MAXK = (NCHUNKS + NW - 1) // NW
GROUPS = CHUNK // 16
CW = 16
NLANES = D // 16
TCB = 1000
NTCB = (N - NSC) // TCB


def _sc_segment_sum_body(z_hbm, batch_hbm, sums_out, counts_out,
                         idx_all, rows_v0, rows_v1, acc_v, cnt_v,
                         sem0, sem1):
    cid = lax.axis_index("c")
    sid = lax.axis_index("s")
    wid = sid * NC + cid

    zeros16 = jnp.zeros((16,), jnp.float32)
    ones16 = jnp.ones((16,), jnp.float32)
    sixteen16 = jnp.full((16,), 16.0, jnp.float32)

    start = (wid * NCHUNKS) // NW
    end = ((wid + 1) * NCHUNKS) // NW
    count = end - start

    rows = (rows_v0, rows_v1)
    sems = (sem0, sem1)

    def dma(c, buf):
        return pltpu.make_async_copy(
            z_hbm.at[pl.ds((start + c) * CHUNK, CHUNK)],
            rows[buf], sems[buf])

    dma(0, 0).start()
    pltpu.sync_copy(batch_hbm.at[pl.ds(start * CHUNK, MAXK * CHUNK)], idx_all)

    def zero_acc(i, c):
        for k in range(NLANES):
            acc_v[i, pl.ds(16 * k, 16)] = zeros16
        return c
    lax.fori_loop(0, B, zero_acc, 0)

    def zero_cnt(i, c):
        for k in range(8):
            cnt_v[8 * i + k, :] = zeros16
        return c
    lax.fori_loop(0, B // 8, zero_cnt, 0)

    def compute(c, rows_v):
        def group_step(g, cc):
            iv = idx_all[pl.ds(c * CHUNK + g * 16, 16)]
            r0 = g * 16
            seg0 = iv[0]
            uniform = seg0 == iv[15]

            @pl.when(uniform)
            def _():
                for h in range(2):
                    cols = range(8 * h, 8 * h + 8)
                    a = [rows_v[r0, pl.ds(16 * i, 16)] for i in cols]
                    for j in range(1, 16):
                        for x, i in enumerate(cols):
                            a[x] = a[x] + rows_v[r0 + j, pl.ds(16 * i, 16)]
                    for x, i in enumerate(cols):
                        sl = pl.ds(16 * i, 16)
                        acc_v[seg0, sl] = acc_v[seg0, sl] + a[x]
                cnt_v[seg0, :] = cnt_v[seg0, :] + sixteen16

            @pl.when(jnp.logical_not(uniform))
            def _():
                for j in range(16):
                    seg = iv[j]
                    for h in range(2):
                        cols = range(8 * h, 8 * h + 8)
                        r = [rows_v[r0 + j, pl.ds(16 * i, 16)]
                             for i in cols]
                        olds = [acc_v[seg, pl.ds(16 * i, 16)]
                                for i in cols]
                        for x, i in enumerate(cols):
                            acc_v[seg, pl.ds(16 * i, 16)] = olds[x] + r[x]
                    cnt_v[seg, :] = cnt_v[seg, :] + ones16
            return cc
        lax.fori_loop(0, GROUPS, group_step, 0)

    def pair_step(k, carry):
        for b in range(2):
            c = 2 * k + b
            nxt = c + 1

            @pl.when(nxt < count)
            def _():
                dma(nxt, 1 - b).start()

            @pl.when(c < count)
            def _():
                dma(c, b).wait()
                compute(c, rows[b])
        return carry

    lax.fori_loop(0, (MAXK + 1) // 2, pair_step, 0)

    pltpu.sync_copy(acc_v, sums_out.at[wid])
    pltpu.sync_copy(cnt_v, counts_out.at[wid])


@functools.partial(
    pl.kernel,
    out_type=[
        jax.ShapeDtypeStruct((NW, B, D), jnp.float32),
        jax.ShapeDtypeStruct((NW, B, CW), jnp.float32),
    ],
    mesh=plsc.VectorSubcoreMesh(core_axis_name="c", subcore_axis_name="s"),
    scratch_types=[
        pltpu.VMEM((MAXK * CHUNK,), jnp.int32),
        pltpu.VMEM((CHUNK, D), jnp.float32),
        pltpu.VMEM((CHUNK, D), jnp.float32),
        pltpu.VMEM((B, D), jnp.float32),
        pltpu.VMEM((B, CW), jnp.float32),
        pltpu.SemaphoreType.DMA,
        pltpu.SemaphoreType.DMA,
    ],
)
def _sc_segment_sum(*refs):
    _sc_segment_sum_body(*refs)


def _tc_partial_body(z_ref, batch_ref, sums_ref, cnt_ref):
    k = pl.program_id(0)

    @pl.when(k == 0)
    def _():
        sums_ref[...] = jnp.zeros_like(sums_ref)
        cnt_ref[...] = jnp.zeros_like(cnt_ref)

    ids = batch_ref[0, 0]
    segs = jax.lax.broadcasted_iota(jnp.int32, (B, TCB), 0)
    onehot = (segs == ids[None, :]).astype(jnp.float32)
    sums_ref[...] += jnp.dot(onehot, z_ref[...],
                             preferred_element_type=jnp.float32,
                             precision=jax.lax.Precision.HIGHEST)
    cnt_ref[...] += jnp.sum(onehot, axis=1)[None, :]


def _tc_merge_body(scp_ref, scc_ref, tcp_ref, tcc_ref,
                   u_ref, w_ref, b_ref, out_ref):
    sums = jnp.sum(scp_ref[...], axis=0) + tcp_ref[...]
    counts = jnp.sum(scc_ref[...], axis=0)[:, 0] + tcc_ref[0]
    graph = sums / jnp.maximum(counts, 1.0)[:, None]
    glob = jnp.dot(u_ref[...], w_ref[...], preferred_element_type=jnp.float32)
    glob = jnp.maximum(glob + b_ref[...], 0.0)
    out_ref[...] = jnp.concatenate([graph, glob], axis=-1)


def kernel(z, u, batch, batch_size, W, b):
    del batch_size
    batch32 = batch.astype(jnp.int32)

    scp, scc = _sc_segment_sum(z, batch32)

    tcp, tcc = pl.pallas_call(
        _tc_partial_body,
        grid=(NTCB,),
        in_specs=[
            pl.BlockSpec((TCB, D), lambda k: (NSC // TCB + k, 0)),
            pl.BlockSpec((1, 1, TCB), lambda k: (NSC // TCB + k, 0, 0)),
        ],
        out_specs=[
            pl.BlockSpec((B, D), lambda k: (0, 0)),
            pl.BlockSpec((1, B), lambda k: (0, 0)),
        ],
        out_shape=[
            jax.ShapeDtypeStruct((B, D), jnp.float32),
            jax.ShapeDtypeStruct((1, B), jnp.float32),
        ],
    )(z, batch32.reshape(N // TCB, 1, TCB))

    out = pl.pallas_call(
        _tc_merge_body,
        out_shape=jax.ShapeDtypeStruct((B, 2 * D), jnp.float32),
    )(scp, scc, tcp, tcc, u, W, b.reshape(1, D))
    return out


# speedup vs baseline: 1.1447x; 1.1447x over previous
import functools

import jax
import jax.numpy as jnp
from jax import lax
from jax.experimental import pallas as pl
from jax.experimental.pallas import tpu as pltpu
from jax.experimental.pallas import tpu_sc as plsc

N, D = 50000, 256
B = 128
NSC = 20000
CHUNK = 80
NCHUNKS = NSC // CHUNK
NC, NS = 2, 16
NW = NC * NS
MAXK = (NCHUNKS + NW - 1) // NW
GROUPS = CHUNK // 16
CW = 16
NLANES = D // 16
TCB = 1000
NTCB = (N - NSC) // TCB


def _sc_segment_sum_body(z_hbm, batch_hbm, sums_out, counts_out,
                         idx_all, rows_v0, rows_v1, acc_v, cnt_v,
                         sem0, sem1):
    cid = lax.axis_index("c")
    sid = lax.axis_index("s")
    wid = sid * NC + cid

    zeros16 = jnp.zeros((16,), jnp.float32)
    ones16 = jnp.ones((16,), jnp.float32)
    sixteen16 = jnp.full((16,), 16.0, jnp.float32)

    start = (wid * NCHUNKS) // NW
    end = ((wid + 1) * NCHUNKS) // NW
    count = end - start

    rows = (rows_v0, rows_v1)
    sems = (sem0, sem1)

    def dma(c, buf):
        return pltpu.make_async_copy(
            z_hbm.at[pl.ds((start + c) * CHUNK, CHUNK)],
            rows[buf], sems[buf])

    dma(0, 0).start()
    pltpu.sync_copy(batch_hbm.at[pl.ds(start * CHUNK, MAXK * CHUNK)], idx_all)

    def zero_acc(i, c):
        for k in range(NLANES):
            acc_v[i, pl.ds(16 * k, 16)] = zeros16
        return c
    lax.fori_loop(0, B, zero_acc, 0)

    def zero_cnt(i, c):
        for k in range(8):
            cnt_v[8 * i + k, :] = zeros16
        return c
    lax.fori_loop(0, B // 8, zero_cnt, 0)

    def compute(c, rows_v):
        def group_step(g, cc):
            iv = idx_all[pl.ds(c * CHUNK + g * 16, 16)]
            r0 = g * 16
            seg0 = iv[0]
            uniform = seg0 == iv[15]

            @pl.when(uniform)
            def _():
                for h in range(2):
                    cols = range(8 * h, 8 * h + 8)
                    a = [rows_v[r0, pl.ds(16 * i, 16)] for i in cols]
                    for j in range(1, 16):
                        for x, i in enumerate(cols):
                            a[x] = a[x] + rows_v[r0 + j, pl.ds(16 * i, 16)]
                    for x, i in enumerate(cols):
                        sl = pl.ds(16 * i, 16)
                        acc_v[seg0, sl] = acc_v[seg0, sl] + a[x]
                cnt_v[seg0, :] = cnt_v[seg0, :] + sixteen16

            @pl.when(jnp.logical_not(uniform))
            def _():
                for j in range(16):
                    seg = iv[j]
                    for h in range(2):
                        cols = range(8 * h, 8 * h + 8)
                        r = [rows_v[r0 + j, pl.ds(16 * i, 16)]
                             for i in cols]
                        olds = [acc_v[seg, pl.ds(16 * i, 16)]
                                for i in cols]
                        for x, i in enumerate(cols):
                            acc_v[seg, pl.ds(16 * i, 16)] = olds[x] + r[x]
                    cnt_v[seg, :] = cnt_v[seg, :] + ones16
            return cc
        lax.fori_loop(0, GROUPS, group_step, 0)

    def pair_step(k, carry):
        for b in range(2):
            c = 2 * k + b
            nxt = c + 1

            @pl.when(nxt < count)
            def _():
                dma(nxt, 1 - b).start()

            @pl.when(c < count)
            def _():
                dma(c, b).wait()
                compute(c, rows[b])
        return carry

    lax.fori_loop(0, (MAXK + 1) // 2, pair_step, 0)

    pltpu.sync_copy(acc_v, sums_out.at[wid])
    pltpu.sync_copy(cnt_v, counts_out.at[wid])


@functools.partial(
    pl.kernel,
    out_type=[
        jax.ShapeDtypeStruct((NW, B, D), jnp.float32),
        jax.ShapeDtypeStruct((NW, B, CW), jnp.float32),
    ],
    mesh=plsc.VectorSubcoreMesh(core_axis_name="c", subcore_axis_name="s"),
    scratch_types=[
        pltpu.VMEM((MAXK * CHUNK,), jnp.int32),
        pltpu.VMEM((CHUNK, D), jnp.float32),
        pltpu.VMEM((CHUNK, D), jnp.float32),
        pltpu.VMEM((B, D), jnp.float32),
        pltpu.VMEM((B, CW), jnp.float32),
        pltpu.SemaphoreType.DMA,
        pltpu.SemaphoreType.DMA,
    ],
)
def _sc_segment_sum(*refs):
    _sc_segment_sum_body(*refs)


def _tc_partial_body(z_ref, batch_ref, sums_ref, cnt_ref):
    k = pl.program_id(0)

    @pl.when(k == 0)
    def _():
        sums_ref[...] = jnp.zeros_like(sums_ref)
        cnt_ref[...] = jnp.zeros_like(cnt_ref)

    ids = batch_ref[0, 0]
    segs = jax.lax.broadcasted_iota(jnp.int32, (B, TCB), 0)
    onehot = (segs == ids[None, :]).astype(jnp.float32)
    sums_ref[...] += jnp.dot(onehot, z_ref[...],
                             preferred_element_type=jnp.float32)
    cnt_ref[...] += jnp.sum(onehot, axis=1)[None, :]


def _tc_merge_body(scp_ref, scc_ref, tcp_ref, tcc_ref,
                   u_ref, w_ref, b_ref, out_ref):
    sums = jnp.sum(scp_ref[...], axis=0) + tcp_ref[...]
    counts = jnp.sum(scc_ref[...], axis=0)[:, 0] + tcc_ref[0]
    graph = sums / jnp.maximum(counts, 1.0)[:, None]
    glob = jnp.dot(u_ref[...], w_ref[...], preferred_element_type=jnp.float32)
    glob = jnp.maximum(glob + b_ref[...], 0.0)
    out_ref[...] = jnp.concatenate([graph, glob], axis=-1)


def kernel(z, u, batch, batch_size, W, b):
    del batch_size
    batch32 = batch.astype(jnp.int32)

    scp, scc = _sc_segment_sum(z, batch32)

    tcp, tcc = pl.pallas_call(
        _tc_partial_body,
        grid=(NTCB,),
        in_specs=[
            pl.BlockSpec((TCB, D), lambda k: (NSC // TCB + k, 0)),
            pl.BlockSpec((1, 1, TCB), lambda k: (NSC // TCB + k, 0, 0)),
        ],
        out_specs=[
            pl.BlockSpec((B, D), lambda k: (0, 0)),
            pl.BlockSpec((1, B), lambda k: (0, 0)),
        ],
        out_shape=[
            jax.ShapeDtypeStruct((B, D), jnp.float32),
            jax.ShapeDtypeStruct((1, B), jnp.float32),
        ],
    )(z, batch32.reshape(N // TCB, 1, TCB))

    out = pl.pallas_call(
        _tc_merge_body,
        out_shape=jax.ShapeDtypeStruct((B, 2 * D), jnp.float32),
    )(scp, scc, tcp, tcc, u, W, b.reshape(1, D))
    return out


# speedup vs baseline: 1.1764x; 1.0277x over previous
import functools

import jax
import jax.numpy as jnp
from jax import lax
from jax.experimental import pallas as pl
from jax.experimental.pallas import tpu as pltpu
from jax.experimental.pallas import tpu_sc as plsc

N, D = 50000, 256
B = 128
NSC = 28000
CHUNK = 80
NCHUNKS = NSC // CHUNK
NC, NS = 2, 16
NW = NC * NS
MAXK = (NCHUNKS + NW - 1) // NW
GROUPS = CHUNK // 16
CW = 16
NLANES = D // 16
TCB = 1000
NTCB = (N - NSC) // TCB


def _sc_segment_sum_body(z_hbm, batch_hbm, sums_out, counts_out,
                         idx_all, rows_v0, rows_v1, acc_v, cnt_v,
                         sem0, sem1):
    cid = lax.axis_index("c")
    sid = lax.axis_index("s")
    wid = sid * NC + cid

    zeros16 = jnp.zeros((16,), jnp.float32)
    ones16 = jnp.ones((16,), jnp.float32)
    sixteen16 = jnp.full((16,), 16.0, jnp.float32)

    start = (wid * NCHUNKS) // NW
    end = ((wid + 1) * NCHUNKS) // NW
    count = end - start

    rows = (rows_v0, rows_v1)
    sems = (sem0, sem1)

    def dma(c, buf):
        return pltpu.make_async_copy(
            z_hbm.at[pl.ds((start + c) * CHUNK, CHUNK)],
            rows[buf], sems[buf])

    dma(0, 0).start()
    pltpu.sync_copy(batch_hbm.at[pl.ds(start * CHUNK, MAXK * CHUNK)], idx_all)

    def zero_acc(i, c):
        for k in range(NLANES):
            acc_v[i, pl.ds(16 * k, 16)] = zeros16
        return c
    lax.fori_loop(0, B, zero_acc, 0)

    def zero_cnt(i, c):
        for k in range(8):
            cnt_v[8 * i + k, :] = zeros16
        return c
    lax.fori_loop(0, B // 8, zero_cnt, 0)

    def compute(c, rows_v):
        def group_step(g, cc):
            iv = idx_all[pl.ds(c * CHUNK + g * 16, 16)]
            r0 = g * 16
            seg0 = iv[0]
            uniform = seg0 == iv[15]

            @pl.when(uniform)
            def _():
                for h in range(2):
                    cols = range(8 * h, 8 * h + 8)
                    a = [rows_v[r0, pl.ds(16 * i, 16)] for i in cols]
                    for j in range(1, 16):
                        for x, i in enumerate(cols):
                            a[x] = a[x] + rows_v[r0 + j, pl.ds(16 * i, 16)]
                    for x, i in enumerate(cols):
                        sl = pl.ds(16 * i, 16)
                        acc_v[seg0, sl] = acc_v[seg0, sl] + a[x]
                cnt_v[seg0, :] = cnt_v[seg0, :] + sixteen16

            @pl.when(jnp.logical_not(uniform))
            def _():
                for j in range(16):
                    seg = iv[j]
                    for h in range(2):
                        cols = range(8 * h, 8 * h + 8)
                        r = [rows_v[r0 + j, pl.ds(16 * i, 16)]
                             for i in cols]
                        olds = [acc_v[seg, pl.ds(16 * i, 16)]
                                for i in cols]
                        for x, i in enumerate(cols):
                            acc_v[seg, pl.ds(16 * i, 16)] = olds[x] + r[x]
                    cnt_v[seg, :] = cnt_v[seg, :] + ones16
            return cc
        lax.fori_loop(0, GROUPS, group_step, 0)

    def pair_step(k, carry):
        for b in range(2):
            c = 2 * k + b
            nxt = c + 1

            @pl.when(nxt < count)
            def _():
                dma(nxt, 1 - b).start()

            @pl.when(c < count)
            def _():
                dma(c, b).wait()
                compute(c, rows[b])
        return carry

    lax.fori_loop(0, (MAXK + 1) // 2, pair_step, 0)

    pltpu.sync_copy(acc_v, sums_out.at[wid])
    pltpu.sync_copy(cnt_v, counts_out.at[wid])


@functools.partial(
    pl.kernel,
    out_type=[
        jax.ShapeDtypeStruct((NW, B, D), jnp.float32),
        jax.ShapeDtypeStruct((NW, B, CW), jnp.float32),
    ],
    mesh=plsc.VectorSubcoreMesh(core_axis_name="c", subcore_axis_name="s"),
    scratch_types=[
        pltpu.VMEM((MAXK * CHUNK,), jnp.int32),
        pltpu.VMEM((CHUNK, D), jnp.float32),
        pltpu.VMEM((CHUNK, D), jnp.float32),
        pltpu.VMEM((B, D), jnp.float32),
        pltpu.VMEM((B, CW), jnp.float32),
        pltpu.SemaphoreType.DMA,
        pltpu.SemaphoreType.DMA,
    ],
)
def _sc_segment_sum(*refs):
    _sc_segment_sum_body(*refs)


def _tc_partial_body(z_ref, batch_ref, sums_ref, cnt_ref):
    k = pl.program_id(0)

    @pl.when(k == 0)
    def _():
        sums_ref[...] = jnp.zeros_like(sums_ref)
        cnt_ref[...] = jnp.zeros_like(cnt_ref)

    ids = batch_ref[0, 0]
    segs = jax.lax.broadcasted_iota(jnp.int32, (B, TCB), 0)
    onehot = (segs == ids[None, :]).astype(jnp.float32)
    sums_ref[...] += jnp.dot(onehot, z_ref[...],
                             preferred_element_type=jnp.float32)
    cnt_ref[...] += jnp.sum(onehot, axis=1)[None, :]


def _tc_merge_body(scp_ref, scc_ref, tcp_ref, tcc_ref,
                   u_ref, w_ref, b_ref, out_ref):
    sums = jnp.sum(scp_ref[...], axis=0) + tcp_ref[...]
    counts = jnp.sum(scc_ref[...], axis=0)[:, 0] + tcc_ref[0]
    graph = sums / jnp.maximum(counts, 1.0)[:, None]
    glob = jnp.dot(u_ref[...], w_ref[...], preferred_element_type=jnp.float32)
    glob = jnp.maximum(glob + b_ref[...], 0.0)
    out_ref[...] = jnp.concatenate([graph, glob], axis=-1)


def kernel(z, u, batch, batch_size, W, b):
    del batch_size
    batch32 = batch.astype(jnp.int32)

    scp, scc = _sc_segment_sum(z, batch32)

    tcp, tcc = pl.pallas_call(
        _tc_partial_body,
        grid=(NTCB,),
        in_specs=[
            pl.BlockSpec((TCB, D), lambda k: (NSC // TCB + k, 0)),
            pl.BlockSpec((1, 1, TCB), lambda k: (NSC // TCB + k, 0, 0)),
        ],
        out_specs=[
            pl.BlockSpec((B, D), lambda k: (0, 0)),
            pl.BlockSpec((1, B), lambda k: (0, 0)),
        ],
        out_shape=[
            jax.ShapeDtypeStruct((B, D), jnp.float32),
            jax.ShapeDtypeStruct((1, B), jnp.float32),
        ],
    )(z, batch32.reshape(N // TCB, 1, TCB))

    out = pl.pallas_call(
        _tc_merge_body,
        out_shape=jax.ShapeDtypeStruct((B, 2 * D), jnp.float32),
    )(scp, scc, tcp, tcc, u, W, b.reshape(1, D))
    return out
